# Initial kernel scaffold; baseline (speedup 1.0000x reference)
import jax
import jax.numpy as jnp
from jax.experimental import pallas as pl


def kernel(predicted_coords, predicted_points, target_coords, target_points):
    raise NotImplementedError("write your pallas kernel here")


# speedup vs baseline: 1.5007x; 1.5007x over previous
import jax
import jax.numpy as jnp
from jax.experimental import pallas as pl
from jax.experimental.pallas import tpu as pltpu

_B, _N, _M, _D = 8, 4096, 4096, 3
_TILE_M = 512
_MI = _M // _TILE_M
_SENTINEL = 1e30


def _nn_loss_kernel(pp_ref, tp_ref, pred_ref, tgt_ref,
                    total_ref, coord_ref, pts_ref,
                    acc_ref, csum_ref):
    b = pl.program_id(0)
    mi = pl.program_id(1)

    p = pred_ref[0]
    t = tgt_ref[0]

    tnorm = jnp.sum(t * t, axis=0, keepdims=True)
    j = mi * _TILE_M + jax.lax.broadcasted_iota(jnp.int32, (1, _TILE_M), 1)
    tnorm = jnp.where(j < tp_ref[b], tnorm, jnp.float32(_SENTINEL))

    rows = jax.lax.broadcasted_iota(jnp.int32, (8, 1), 0)
    lhs = jnp.where(rows == 3, jnp.float32(1.0), -2.0 * p)
    rhs = jnp.where(rows == 3, tnorm, t)
    cand = jax.lax.dot_general(
        lhs, rhs, (((0,), (0,)), ((), ())),
        preferred_element_type=jnp.float32,
        precision=jax.lax.Precision.HIGHEST)

    @pl.when(mi == 0)
    def _():
        acc_ref[...] = cand

    @pl.when(mi != 0)
    def _():
        acc_ref[...] = jnp.minimum(acc_ref[...], cand)

    @pl.when(mi == _MI - 1)
    def _():
        rowmin = jnp.min(acc_ref[...], axis=1)
        pnorm = jnp.sum(p * p, axis=0)
        i = jax.lax.iota(jnp.int32, _N)
        pcount = pp_ref[b]
        vals = jnp.where(i < pcount, rowmin + pnorm, jnp.float32(0.0))
        s = jnp.sum(vals) / (pcount.astype(jnp.float32) * jnp.float32(_D))

        @pl.when(b == 0)
        def _():
            csum_ref[0, 0] = s

        @pl.when(b != 0)
        def _():
            csum_ref[0, 0] = csum_ref[0, 0] + s

        @pl.when(b == _B - 1)
        def _():
            coord = csum_ref[0, 0] / jnp.float32(_B)
            pts = jnp.float32(0.0)
            for bb in range(_B):
                dv = (pp_ref[bb] - tp_ref[bb]).astype(jnp.float32) / _N
                pts = pts + dv * dv
            pts = pts / jnp.float32(_B)
            coord_ref[0, 0] = coord
            pts_ref[0, 0] = pts
            total_ref[0, 0] = coord + jnp.float32(0.1) * pts


def kernel(predicted_coords, predicted_points, target_coords, target_points):
    pp = predicted_points.astype(jnp.int32)
    tp = target_points.astype(jnp.int32)
    pred_t = jnp.pad(jnp.transpose(predicted_coords, (0, 2, 1)),
                     ((0, 0), (0, 8 - _D), (0, 0)))
    tgt_t = jnp.pad(jnp.transpose(target_coords, (0, 2, 1)),
                    ((0, 0), (0, 8 - _D), (0, 0)))

    out_shape = [jax.ShapeDtypeStruct((1, 1), jnp.float32)] * 3
    total, coord, pts = pl.pallas_call(
        _nn_loss_kernel,
        grid=(_B, _MI),
        in_specs=[
            pl.BlockSpec(memory_space=pltpu.SMEM),
            pl.BlockSpec(memory_space=pltpu.SMEM),
            pl.BlockSpec((1, 8, _N), lambda b, mi: (b, 0, 0)),
            pl.BlockSpec((1, 8, _TILE_M), lambda b, mi: (b, 0, mi)),
        ],
        out_specs=[
            pl.BlockSpec(memory_space=pltpu.SMEM),
            pl.BlockSpec(memory_space=pltpu.SMEM),
            pl.BlockSpec(memory_space=pltpu.SMEM),
        ],
        out_shape=out_shape,
        scratch_shapes=[
            pltpu.VMEM((_N, _TILE_M), jnp.float32),
            pltpu.SMEM((1, 1), jnp.float32),
        ],
        compiler_params=pltpu.CompilerParams(
            dimension_semantics=("arbitrary", "arbitrary")),
    )(pp, tp, pred_t, tgt_t)
    return total[0, 0], coord[0, 0], pts[0, 0]


# speedup vs baseline: 6.3373x; 4.2230x over previous
import jax
import jax.numpy as jnp
from jax.experimental import pallas as pl
from jax.experimental.pallas import tpu as pltpu

_B, _N, _M, _D = 8, 4096, 4096, 3
_TILE_M = 512
_MI = _M // _TILE_M
_SENTINEL = 1e30


def _nn_loss_kernel(pp_ref, tp_ref, pred_ref, tgt_ref,
                    total_ref, coord_ref, pts_ref,
                    acc_ref, csum_ref):
    b = pl.program_id(0)
    mi = pl.program_id(1)

    p = pred_ref[0]
    t = tgt_ref[0]

    tnorm = jnp.sum(t * t, axis=0, keepdims=True)
    j = mi * _TILE_M + jax.lax.broadcasted_iota(jnp.int32, (1, _TILE_M), 1)
    tnorm = jnp.where(j < tp_ref[b], tnorm, jnp.float32(_SENTINEL))

    rows = jax.lax.broadcasted_iota(jnp.int32, (8, 1), 0)
    lhs = jnp.where(rows == 3, jnp.float32(1.0), -2.0 * p)
    rhs = jnp.where(rows == 3, tnorm, t)
    lhs_hi = lhs.astype(jnp.bfloat16)
    lhs_lo = (lhs - lhs_hi.astype(jnp.float32)).astype(jnp.bfloat16)
    rhs_hi = rhs.astype(jnp.bfloat16)
    rhs_lo = (rhs - rhs_hi.astype(jnp.float32)).astype(jnp.bfloat16)
    lhs24 = jnp.concatenate([lhs_hi, lhs_hi, lhs_lo], axis=0)
    rhs24 = jnp.concatenate([rhs_hi, rhs_lo, rhs_hi], axis=0)
    cand = jax.lax.dot_general(
        lhs24, rhs24, (((0,), (0,)), ((), ())),
        preferred_element_type=jnp.float32)

    m = cand[:, 0:128]
    for k in range(1, _TILE_M // 128):
        m = jnp.minimum(m, cand[:, k * 128:(k + 1) * 128])

    @pl.when(mi == 0)
    def _():
        acc_ref[...] = m

    @pl.when(mi != 0)
    def _():
        acc_ref[...] = jnp.minimum(acc_ref[...], m)

    @pl.when(mi == _MI - 1)
    def _():
        rowmin = jnp.min(acc_ref[...], axis=1)
        pnorm = jnp.sum(p * p, axis=0)
        i = jax.lax.iota(jnp.int32, _N)
        pcount = pp_ref[b]
        vals = jnp.where(i < pcount, rowmin + pnorm, jnp.float32(0.0))
        s = jnp.sum(vals) / (pcount.astype(jnp.float32) * jnp.float32(_D))

        @pl.when(b == 0)
        def _():
            csum_ref[0, 0] = s

        @pl.when(b != 0)
        def _():
            csum_ref[0, 0] = csum_ref[0, 0] + s

        @pl.when(b == _B - 1)
        def _():
            coord = csum_ref[0, 0] / jnp.float32(_B)
            pts = jnp.float32(0.0)
            for bb in range(_B):
                dv = (pp_ref[bb] - tp_ref[bb]).astype(jnp.float32) / _N
                pts = pts + dv * dv
            pts = pts / jnp.float32(_B)
            coord_ref[0, 0] = coord
            pts_ref[0, 0] = pts
            total_ref[0, 0] = coord + jnp.float32(0.1) * pts


def kernel(predicted_coords, predicted_points, target_coords, target_points):
    pp = predicted_points.astype(jnp.int32)
    tp = target_points.astype(jnp.int32)
    pred_t = jnp.pad(jnp.transpose(predicted_coords, (0, 2, 1)),
                     ((0, 0), (0, 8 - _D), (0, 0)))
    tgt_t = jnp.pad(jnp.transpose(target_coords, (0, 2, 1)),
                    ((0, 0), (0, 8 - _D), (0, 0)))

    out_shape = [jax.ShapeDtypeStruct((1, 1), jnp.float32)] * 3
    total, coord, pts = pl.pallas_call(
        _nn_loss_kernel,
        grid=(_B, _MI),
        in_specs=[
            pl.BlockSpec(memory_space=pltpu.SMEM),
            pl.BlockSpec(memory_space=pltpu.SMEM),
            pl.BlockSpec((1, 8, _N), lambda b, mi: (b, 0, 0)),
            pl.BlockSpec((1, 8, _TILE_M), lambda b, mi: (b, 0, mi)),
        ],
        out_specs=[
            pl.BlockSpec(memory_space=pltpu.SMEM),
            pl.BlockSpec(memory_space=pltpu.SMEM),
            pl.BlockSpec(memory_space=pltpu.SMEM),
        ],
        out_shape=out_shape,
        scratch_shapes=[
            pltpu.VMEM((_N, 128), jnp.float32),
            pltpu.SMEM((1, 1), jnp.float32),
        ],
        compiler_params=pltpu.CompilerParams(
            dimension_semantics=("arbitrary", "arbitrary")),
    )(pp, tp, pred_t, tgt_t)
    return total[0, 0], coord[0, 0], pts[0, 0]


# speedup vs baseline: 7.2202x; 1.1393x over previous
import jax
import jax.numpy as jnp
from jax.experimental import pallas as pl
from jax.experimental.pallas import tpu as pltpu

_B, _N, _M, _D = 8, 4096, 4096, 3
_TILE_M = 1024
_MI = _M // _TILE_M
_SENTINEL = 1e30


def _nn_loss_kernel(pp_ref, tp_ref, pred_ref, tgt_ref,
                    total_ref, coord_ref, pts_ref,
                    acc_ref, csum_ref):
    b = pl.program_id(0)
    mi = pl.program_id(1)

    p = pred_ref[0]
    t = tgt_ref[0]

    tnorm = jnp.sum(t * t, axis=0, keepdims=True)
    j = mi * _TILE_M + jax.lax.broadcasted_iota(jnp.int32, (1, _TILE_M), 1)
    tnorm = jnp.where(j < tp_ref[b], tnorm, jnp.float32(_SENTINEL))

    rows = jax.lax.broadcasted_iota(jnp.int32, (8, 1), 0)
    lhs = jnp.where(rows == 3, jnp.float32(1.0), -2.0 * p)
    rhs = jnp.where(rows == 3, tnorm, t)
    lhs_hi = lhs.astype(jnp.bfloat16)
    lhs_lo = (lhs - lhs_hi.astype(jnp.float32)).astype(jnp.bfloat16)
    rhs_hi = rhs.astype(jnp.bfloat16)
    rhs_lo = (rhs - rhs_hi.astype(jnp.float32)).astype(jnp.bfloat16)
    lhs24 = jnp.concatenate([lhs_hi, lhs_hi, lhs_lo], axis=0)
    rhs24 = jnp.concatenate([rhs_hi, rhs_lo, rhs_hi], axis=0)
    cand = jax.lax.dot_general(
        lhs24, rhs24, (((0,), (0,)), ((), ())),
        preferred_element_type=jnp.float32)

    m = cand[:, 0:128]
    for k in range(1, _TILE_M // 128):
        m = jnp.minimum(m, cand[:, k * 128:(k + 1) * 128])

    @pl.when(mi == 0)
    def _():
        acc_ref[...] = m

    @pl.when(mi != 0)
    def _():
        acc_ref[...] = jnp.minimum(acc_ref[...], m)

    @pl.when(mi == _MI - 1)
    def _():
        rowmin = jnp.min(acc_ref[...], axis=1)
        pnorm = jnp.sum(p * p, axis=0)
        i = jax.lax.iota(jnp.int32, _N)
        pcount = pp_ref[b]
        vals = jnp.where(i < pcount, rowmin + pnorm, jnp.float32(0.0))
        s = jnp.sum(vals) / (pcount.astype(jnp.float32) * jnp.float32(_D))

        @pl.when(b == 0)
        def _():
            csum_ref[0, 0] = s

        @pl.when(b != 0)
        def _():
            csum_ref[0, 0] = csum_ref[0, 0] + s

        @pl.when(b == _B - 1)
        def _():
            coord = csum_ref[0, 0] / jnp.float32(_B)
            pts = jnp.float32(0.0)
            for bb in range(_B):
                dv = (pp_ref[bb] - tp_ref[bb]).astype(jnp.float32) / _N
                pts = pts + dv * dv
            pts = pts / jnp.float32(_B)
            coord_ref[0, 0] = coord
            pts_ref[0, 0] = pts
            total_ref[0, 0] = coord + jnp.float32(0.1) * pts


def kernel(predicted_coords, predicted_points, target_coords, target_points):
    pp = predicted_points.astype(jnp.int32)
    tp = target_points.astype(jnp.int32)
    pred_t = jnp.pad(jnp.transpose(predicted_coords, (0, 2, 1)),
                     ((0, 0), (0, 8 - _D), (0, 0)))
    tgt_t = jnp.pad(jnp.transpose(target_coords, (0, 2, 1)),
                    ((0, 0), (0, 8 - _D), (0, 0)))

    out_shape = [jax.ShapeDtypeStruct((1, 1), jnp.float32)] * 3
    total, coord, pts = pl.pallas_call(
        _nn_loss_kernel,
        grid=(_B, _MI),
        in_specs=[
            pl.BlockSpec(memory_space=pltpu.SMEM),
            pl.BlockSpec(memory_space=pltpu.SMEM),
            pl.BlockSpec((1, 8, _N), lambda b, mi: (b, 0, 0)),
            pl.BlockSpec((1, 8, _TILE_M), lambda b, mi: (b, 0, mi)),
        ],
        out_specs=[
            pl.BlockSpec(memory_space=pltpu.SMEM),
            pl.BlockSpec(memory_space=pltpu.SMEM),
            pl.BlockSpec(memory_space=pltpu.SMEM),
        ],
        out_shape=out_shape,
        scratch_shapes=[
            pltpu.VMEM((_N, 128), jnp.float32),
            pltpu.SMEM((1, 1), jnp.float32),
        ],
        compiler_params=pltpu.CompilerParams(
            dimension_semantics=("arbitrary", "arbitrary")),
    )(pp, tp, pred_t, tgt_t)
    return total[0, 0], coord[0, 0], pts[0, 0]


# speedup vs baseline: 7.7689x; 1.0760x over previous
import jax
import jax.numpy as jnp
from jax.experimental import pallas as pl
from jax.experimental.pallas import tpu as pltpu

_B, _N, _M, _D = 8, 4096, 4096, 3
_TILE_M = 2048
_MI = _M // _TILE_M
_SENTINEL = 1e30


def _nn_loss_kernel(pp_ref, tp_ref, pred_ref, tgt_ref,
                    total_ref, coord_ref, pts_ref,
                    acc_ref, csum_ref):
    b = pl.program_id(0)
    mi = pl.program_id(1)

    p = pred_ref[0]
    t = tgt_ref[0]

    tnorm = jnp.sum(t * t, axis=0, keepdims=True)
    j = mi * _TILE_M + jax.lax.broadcasted_iota(jnp.int32, (1, _TILE_M), 1)
    tnorm = jnp.where(j < tp_ref[b], tnorm, jnp.float32(_SENTINEL))

    rows = jax.lax.broadcasted_iota(jnp.int32, (8, 1), 0)
    lhs = jnp.where(rows == 3, jnp.float32(1.0), -2.0 * p)
    rhs = jnp.where(rows == 3, tnorm, t)
    lhs_hi = lhs.astype(jnp.bfloat16)
    lhs_lo = (lhs - lhs_hi.astype(jnp.float32)).astype(jnp.bfloat16)
    rhs_hi = rhs.astype(jnp.bfloat16)
    rhs_lo = (rhs - rhs_hi.astype(jnp.float32)).astype(jnp.bfloat16)
    lhs24 = jnp.concatenate([lhs_hi, lhs_hi, lhs_lo], axis=0)
    rhs24 = jnp.concatenate([rhs_hi, rhs_lo, rhs_hi], axis=0)
    cand = jax.lax.dot_general(
        lhs24, rhs24, (((0,), (0,)), ((), ())),
        preferred_element_type=jnp.float32)

    m = cand[:, 0:128]
    for k in range(1, _TILE_M // 128):
        m = jnp.minimum(m, cand[:, k * 128:(k + 1) * 128])

    @pl.when(mi == 0)
    def _():
        acc_ref[...] = m

    @pl.when(mi != 0)
    def _():
        acc_ref[...] = jnp.minimum(acc_ref[...], m)

    @pl.when(mi == _MI - 1)
    def _():
        rowmin = jnp.min(acc_ref[...], axis=1)
        pnorm = jnp.sum(p * p, axis=0)
        i = jax.lax.iota(jnp.int32, _N)
        pcount = pp_ref[b]
        vals = jnp.where(i < pcount, rowmin + pnorm, jnp.float32(0.0))
        s = jnp.sum(vals) / (pcount.astype(jnp.float32) * jnp.float32(_D))

        @pl.when(b == 0)
        def _():
            csum_ref[0, 0] = s

        @pl.when(b != 0)
        def _():
            csum_ref[0, 0] = csum_ref[0, 0] + s

        @pl.when(b == _B - 1)
        def _():
            coord = csum_ref[0, 0] / jnp.float32(_B)
            pts = jnp.float32(0.0)
            for bb in range(_B):
                dv = (pp_ref[bb] - tp_ref[bb]).astype(jnp.float32) / _N
                pts = pts + dv * dv
            pts = pts / jnp.float32(_B)
            coord_ref[0, 0] = coord
            pts_ref[0, 0] = pts
            total_ref[0, 0] = coord + jnp.float32(0.1) * pts


def kernel(predicted_coords, predicted_points, target_coords, target_points):
    pp = predicted_points.astype(jnp.int32)
    tp = target_points.astype(jnp.int32)
    pred_t = jnp.pad(jnp.transpose(predicted_coords, (0, 2, 1)),
                     ((0, 0), (0, 8 - _D), (0, 0)))
    tgt_t = jnp.pad(jnp.transpose(target_coords, (0, 2, 1)),
                    ((0, 0), (0, 8 - _D), (0, 0)))

    out_shape = [jax.ShapeDtypeStruct((1, 1), jnp.float32)] * 3
    total, coord, pts = pl.pallas_call(
        _nn_loss_kernel,
        grid=(_B, _MI),
        in_specs=[
            pl.BlockSpec(memory_space=pltpu.SMEM),
            pl.BlockSpec(memory_space=pltpu.SMEM),
            pl.BlockSpec((1, 8, _N), lambda b, mi: (b, 0, 0)),
            pl.BlockSpec((1, 8, _TILE_M), lambda b, mi: (b, 0, mi)),
        ],
        out_specs=[
            pl.BlockSpec(memory_space=pltpu.SMEM),
            pl.BlockSpec(memory_space=pltpu.SMEM),
            pl.BlockSpec(memory_space=pltpu.SMEM),
        ],
        out_shape=out_shape,
        scratch_shapes=[
            pltpu.VMEM((_N, 128), jnp.float32),
            pltpu.SMEM((1, 1), jnp.float32),
        ],
        compiler_params=pltpu.CompilerParams(
            dimension_semantics=("arbitrary", "arbitrary")),
    )(pp, tp, pred_t, tgt_t)
    return total[0, 0], coord[0, 0], pts[0, 0]
